# baseline (speedup 1.0000x reference)
import functools
import numpy as np
import jax
import jax.numpy as jnp
from jax import lax
from jax.experimental import pallas as pl
from jax.experimental.pallas import tpu as pltpu
from jax.experimental.pallas import tpu_sc as plsc

N = 10000
NP = 10112
E = 320000
CH = 128
NCH = 2560
EPAD = NCH * CH - E
NW = 32
CPW = NCH // NW
RPT = NP // 16

_mesh = plsc.VectorSubcoreMesh(core_axis_name="c", subcore_axis_name="s")

_PAD_SRC = np.arange(EPAD, dtype=np.int32) * 13 % N
_PAD_DST = N + (np.arange(EPAD, dtype=np.int32) % (NP - N))


def _pad_edges(ei):
    src = jnp.concatenate([ei[0], jnp.asarray(_PAD_SRC)]).reshape(NCH, CH)
    dst = jnp.concatenate([ei[1], jnp.asarray(_PAD_DST)]).reshape(NCH, CH)
    return src, dst


IB = 16
NB = CPW // IB
NPD = 10240
CPW3 = 3 * NCH // NW
RPT3 = 3 * NPD // 16


def _zero_vec16(ref):
    z = jnp.zeros((16,), jnp.float32)
    for r in range(16):
        for k in range(ref.shape[1] // 16):
            ref[r, pl.ds(k * 16, 16)] = z


RPTD = NPD // 16


@functools.partial(
    pl.kernel,
    out_type=(jax.ShapeDtypeStruct((3 * NPD,), jnp.float32),
              jax.ShapeDtypeStruct((3 * NPD,), jnp.float32)),
    mesh=_mesh,
    scratch_types=dict(
        acc1=pltpu.VMEM_SHARED((NPD,), jnp.float32),
        acc2=pltpu.VMEM_SHARED((NPD,), jnp.float32),
        acc3=pltpu.VMEM_SHARED((NPD,), jnp.float32),
        didx=pltpu.VMEM((CPW, CH), jnp.int32),
        ones=pltpu.VMEM((CH,), jnp.float32),
        zbuf=pltpu.VMEM((RPTD,), jnp.float32),
        sem=pltpu.SemaphoreType.DMA,
        semi=pltpu.SemaphoreType.DMA,
    ),
)
def _sc_degree(dst1m, dst2m, dst3m, outa, outb, acc1, acc2, acc3, didx,
               ones, zbuf, sem, semi):
    c = lax.axis_index("c")
    s = lax.axis_index("s")
    wid = s * 2 + c
    accs = (acc1, acc2, acc3)
    dsts = (dst1m, dst2m, dst3m)
    one = jnp.ones((16,), jnp.float32)
    zero = jnp.zeros((16,), jnp.float32)
    pltpu.async_copy(dst1m.at[pl.ds(wid * CPW, CPW)], didx, semi)
    for k in range(CH // 16):
        ones[pl.ds(k * 16, 16)] = one
    for k in range(RPTD // 16):
        zbuf[pl.ds(k * 16, 16)] = zero
    for acc in accs:
        pltpu.sync_copy(zbuf, acc.at[pl.ds(s * RPTD, RPTD)])
    pltpu.make_async_copy(dst1m.at[pl.ds(wid * CPW, CPW)], didx,
                          semi).wait()
    plsc.subcore_barrier()

    KF = 8

    for b, acc in enumerate(accs):
        def body(jj, _, acc=acc):
            for i in range(KF):
                pltpu.async_copy(ones, acc.at[didx.at[jj * KF + i]], sem,
                                 add=True)
            for i in range(KF):
                pltpu.make_async_copy(ones, acc.at[didx.at[jj * KF + i]],
                                      sem).wait()
            return 0

        lax.fori_loop(0, CPW // KF, body, 0)
        if b + 1 < 3:
            pltpu.sync_copy(dsts[b + 1].at[pl.ds(wid * CPW, CPW)], didx)

    plsc.subcore_barrier()

    @pl.when(c == 0)
    def _():
        for b, acc in enumerate(accs):
            pltpu.sync_copy(acc.at[pl.ds(s * RPTD, RPTD)],
                            outa.at[pl.ds(b * NPD + s * RPTD, RPTD)])

    @pl.when(c == 1)
    def _():
        for b, acc in enumerate(accs):
            pltpu.sync_copy(acc.at[pl.ds(s * RPTD, RPTD)],
                            outb.at[pl.ds(b * NPD + s * RPTD, RPTD)])


def _make_sc_agg(D):
    @functools.partial(
        pl.kernel,
        out_type=jax.ShapeDtypeStruct((2, NP, D), jnp.float32),
        mesh=_mesh,
        scratch_types=dict(
            acc=pltpu.VMEM_SHARED((NP, D), jnp.float32),
            sidx0=pltpu.VMEM((IB, CH), jnp.int32),
            sidx1=pltpu.VMEM((IB, CH), jnp.int32),
            didx0=pltpu.VMEM((IB, CH), jnp.int32),
            didx1=pltpu.VMEM((IB, CH), jnp.int32),
            rows0=pltpu.VMEM((CH, D), jnp.float32),
            rows1=pltpu.VMEM((CH, D), jnp.float32),
            semi=pltpu.SemaphoreType.DMA,
            semg0=pltpu.SemaphoreType.DMA,
            semg1=pltpu.SemaphoreType.DMA,
            sems0=pltpu.SemaphoreType.DMA,
            sems1=pltpu.SemaphoreType.DMA,
        ),
    )
    def agg(table, srcm, dstm, out, acc, sidx0, sidx1, didx0, didx1,
            rows0, rows1, semi, semg0, semg1, sems0, sems1):
        c = lax.axis_index("c")
        s = lax.axis_index("s")
        wid = s * 2 + c
        rows = (rows0, rows1)
        semg = (semg0, semg1)
        sems = (sems0, sems1)
        sidx = (sidx0, sidx1)
        didx = (didx0, didx1)
        pltpu.async_copy(srcm.at[pl.ds(wid * CPW, IB)], sidx0, semi)
        pltpu.async_copy(dstm.at[pl.ds(wid * CPW, IB)], didx0, semi)
        z = jnp.zeros((16,), jnp.float32)
        for r in range(CH):
            for k in range(D // 16):
                rows0[r, pl.ds(k * 16, 16)] = z
        for k in range(4):
            pltpu.async_copy(rows0, acc.at[pl.ds(s * RPT + k * CH, CH)],
                             sems0)
        pltpu.async_copy(rows0.at[pl.ds(0, RPT - 4 * CH)],
                         acc.at[pl.ds(s * RPT + 4 * CH, RPT - 4 * CH)],
                         sems0)
        for k in range(4):
            pltpu.make_async_copy(rows0, acc.at[pl.ds(0, CH)], sems0).wait()
        pltpu.make_async_copy(rows0.at[pl.ds(0, RPT - 4 * CH)],
                              acc.at[pl.ds(0, RPT - 4 * CH)], sems0).wait()
        pltpu.make_async_copy(srcm.at[pl.ds(wid * CPW, IB)], sidx0,
                              semi).wait()
        pltpu.make_async_copy(dstm.at[pl.ds(wid * CPW, IB)], didx0,
                              semi).wait()
        plsc.subcore_barrier()

        def drain_s(p):
            pltpu.make_async_copy(rows[p], acc.at[didx0.at[0]],
                                  sems[p]).wait()

        def wait_g(p, si, q):
            pltpu.make_async_copy(table.at[si.at[q]], rows[p],
                                  semg[p]).wait()

        pltpu.async_copy(table.at[sidx0.at[0]], rows0, semg0)

        for bb in range(NB):
            pb = bb % 2
            nb = 1 - pb
            si, di = sidx[pb], didx[pb]
            if bb + 1 < NB:
                off = wid * CPW + (bb + 1) * IB
                pltpu.async_copy(srcm.at[pl.ds(off, IB)], sidx[nb], semi)
                pltpu.async_copy(dstm.at[pl.ds(off, IB)], didx[nb], semi)

            def body(jj, _, si=si, di=di, first=(bb == 0)):
                for p in range(2):
                    q = jj * 2 + p

                    def drain(p=p):
                        drain_s(1 - p)

                    def prefetch(p=p, q=q, si=si):
                        pltpu.async_copy(table.at[si.at[q + 1]],
                                         rows[1 - p], semg[1 - p])

                    if p == 0:
                        if first:
                            pl.when(jj > 0)(drain)
                        else:
                            drain()
                        prefetch()
                    else:
                        drain()
                        prefetch()
                    wait_g(p, si, q)
                    pltpu.async_copy(rows[p], acc.at[di.at[q]], sems[p],
                                     add=True)
                return 0

            lax.fori_loop(0, IB // 2 - 1, body, 0)

            drain_s(1)
            pltpu.async_copy(table.at[si.at[IB - 1]], rows1, semg1)
            wait_g(0, si, IB - 2)
            pltpu.async_copy(rows0, acc.at[di.at[IB - 2]], sems0, add=True)

            drain_s(0)
            if bb + 1 < NB:
                off = wid * CPW + (bb + 1) * IB
                pltpu.make_async_copy(srcm.at[pl.ds(off, IB)], sidx[nb],
                                      semi).wait()
                pltpu.make_async_copy(dstm.at[pl.ds(off, IB)], didx[nb],
                                      semi).wait()
                pltpu.async_copy(table.at[sidx[nb].at[0]], rows0, semg0)
            wait_g(1, si, IB - 1)
            pltpu.async_copy(rows1, acc.at[di.at[IB - 1]], sems1, add=True)

        drain_s(1)
        plsc.subcore_barrier()
        pltpu.sync_copy(acc.at[pl.ds(s * RPT, RPT)],
                        out.at[c, pl.ds(s * RPT, RPT)])

    return agg


_sc_agg128 = _make_sc_agg(128)


_BLK = 1264
_GRID = NP // _BLK


def _dinv_body(ca_ref, cb_ref, out_ref):
    deg = ca_ref[...] + cb_ref[...] + 1.0
    out_ref[...] = lax.rsqrt(deg[:, :NP]).reshape(3, NP, 1)


def _dinv(ca, cb):
    return pl.pallas_call(
        _dinv_body,
        out_shape=jax.ShapeDtypeStruct((3, NP, 1), jnp.float32),
    )(ca.reshape(3, NPD), cb.reshape(3, NPD))


def _scale_body(dinv_ref, x_ref, out_ref):
    out_ref[...] = dinv_ref[...] * x_ref[...]


def _scale_rows(dinv, x):
    D = x.shape[1]
    return pl.pallas_call(
        _scale_body,
        grid=(_GRID,),
        in_specs=[
            pl.BlockSpec((_BLK, 1), lambda i: (i, 0)),
            pl.BlockSpec((_BLK, D), lambda i: (i, 0)),
        ],
        out_specs=pl.BlockSpec((_BLK, D), lambda i: (i, 0)),
        out_shape=jax.ShapeDtypeStruct((NP, D), jnp.float32),
    )(dinv, x)


def _layer_body(agg_ref, yp_ref, dinv_ref, wa_ref, ba_ref, wb_ref, out_ref):
    dinv = dinv_ref[...]
    t = dinv * (agg_ref[0] + agg_ref[1] + yp_ref[...])
    h = jax.nn.relu(jnp.dot(t, wa_ref[...],
                            preferred_element_type=jnp.float32) + ba_ref[...])
    out_ref[...] = dinv * jnp.dot(h, wb_ref[...],
                                  preferred_element_type=jnp.float32)


def _layer(agg, yp, dinv, Wa, ba, Wb):
    Din = yp.shape[1]
    Dh = Wa.shape[1]
    Dout = Wb.shape[1]
    return pl.pallas_call(
        _layer_body,
        grid=(_GRID,),
        in_specs=[
            pl.BlockSpec((2, _BLK, Din), lambda i: (0, i, 0)),
            pl.BlockSpec((_BLK, Din), lambda i: (i, 0)),
            pl.BlockSpec((_BLK, 1), lambda i: (i, 0)),
            pl.BlockSpec((Din, Dh), lambda i: (0, 0)),
            pl.BlockSpec((1, Dh), lambda i: (0, 0)),
            pl.BlockSpec((Dh, Dout), lambda i: (0, 0)),
        ],
        out_specs=pl.BlockSpec((_BLK, Dout), lambda i: (i, 0)),
        out_shape=jax.ShapeDtypeStruct((NP, Dout), jnp.float32),
    )(agg, yp, dinv, Wa, ba, Wb)


def _layer2_body(agg_ref, yp_ref, dinv_ref, ba_ref, wb_ref, out_ref):
    dinv = dinv_ref[...]
    h = jax.nn.relu(dinv * (agg_ref[0] + agg_ref[1] + yp_ref[...])
                    + ba_ref[...])
    out_ref[...] = dinv * jnp.dot(h, wb_ref[...],
                                  preferred_element_type=jnp.float32)


def _layer2(agg, yp, dinv, ba, Wb):
    Din = yp.shape[1]
    Dout = Wb.shape[1]
    return pl.pallas_call(
        _layer2_body,
        grid=(_GRID,),
        in_specs=[
            pl.BlockSpec((2, _BLK, Din), lambda i: (0, i, 0)),
            pl.BlockSpec((_BLK, Din), lambda i: (i, 0)),
            pl.BlockSpec((_BLK, 1), lambda i: (i, 0)),
            pl.BlockSpec((1, Din), lambda i: (0, 0)),
            pl.BlockSpec((Din, Dout), lambda i: (0, 0)),
        ],
        out_specs=pl.BlockSpec((_BLK, Dout), lambda i: (i, 0)),
        out_shape=jax.ShapeDtypeStruct((NP, Dout), jnp.float32),
    )(agg, yp, dinv, ba, Wb)


def _final_body(agg_ref, yp_ref, dinv_ref, b_ref, out_ref):
    i = pl.program_id(0)
    o = jax.nn.relu(dinv_ref[...] * (agg_ref[0] + agg_ref[1] + yp_ref[...])
                    + b_ref[...])
    row = i * _BLK + lax.broadcasted_iota(jnp.int32, (_BLK, 1), 0)
    o = jnp.where(row < N, o, 0.0)
    part = jnp.sum(o, axis=0, keepdims=True)

    @pl.when(i == 0)
    def _():
        out_ref[...] = jnp.zeros_like(out_ref)

    out_ref[...] += part


def _final(agg, yp, dinv, b3):
    Dout = yp.shape[1]
    return pl.pallas_call(
        _final_body,
        grid=(_GRID,),
        in_specs=[
            pl.BlockSpec((2, _BLK, Dout), lambda i: (0, i, 0)),
            pl.BlockSpec((_BLK, Dout), lambda i: (i, 0)),
            pl.BlockSpec((_BLK, 1), lambda i: (i, 0)),
            pl.BlockSpec((1, Dout), lambda i: (0, 0)),
        ],
        out_specs=pl.BlockSpec((1, Dout), lambda i: (0, 0)),
        out_shape=jax.ShapeDtypeStruct((1, Dout), jnp.float32),
    )(agg, yp, dinv, b3)


def kernel(x1, edge_index1, x2, edge_index2, x3, edge_index3,
           W1, b1, W2, b2, W3, b3):
    b1r = b1.reshape(1, -1)
    b2r = b2.reshape(1, -1)
    W3p = jnp.pad(W3, ((0, 0), (0, 64)))
    b3r = jnp.pad(b3, (0, 64)).reshape(1, -1)
    pads = [_pad_edges(ei) for ei in (edge_index1, edge_index2, edge_index3)]
    ca, cb = _sc_degree(pads[0][1], pads[1][1], pads[2][1])
    dinv = _dinv(ca, cb)

    total = jnp.zeros((1, 128), jnp.float32)
    for bidx, x in enumerate((x1, x2, x3)):
        src, dst = pads[bidx]
        dv = lax.slice(dinv, (bidx, 0, 0), (bidx + 1, NP, 1)).reshape(NP, 1)
        xp = jnp.pad(x, ((0, NP - N), (0, 0)))
        xs = _scale_rows(dv, xp)
        agg1 = _sc_agg128(xs, src, dst)
        yp2 = _layer(agg1, xs, dv, W1, b1r, W2)
        agg2 = _sc_agg128(yp2, src, dst)
        yp3 = _layer2(agg2, yp2, dv, b2r, W3p)
        agg3 = _sc_agg128(yp3, src, dst)
        total = total + _final(agg3, yp3, dv, b3r)
    return jnp.sum(total) / 64.0


# speedup vs baseline: 1.0185x; 1.0185x over previous
import functools
import numpy as np
import jax
import jax.numpy as jnp
from jax import lax
from jax.experimental import pallas as pl
from jax.experimental.pallas import tpu as pltpu
from jax.experimental.pallas import tpu_sc as plsc

N = 10000
NP = 10112
E = 320000
CH = 128
NCH = 2560
EPAD = NCH * CH - E
NW = 32
CPW = NCH // NW
RPT = NP // 16

_mesh = plsc.VectorSubcoreMesh(core_axis_name="c", subcore_axis_name="s")

_PAD_SRC = np.arange(EPAD, dtype=np.int32) * 13 % N
_PAD_DST = N + (np.arange(EPAD, dtype=np.int32) % (NP - N))


def _pad_edges(ei):
    src = jnp.concatenate([ei[0], jnp.asarray(_PAD_SRC)]).reshape(NCH, CH)
    dst = jnp.concatenate([ei[1], jnp.asarray(_PAD_DST)]).reshape(NCH, CH)
    return src, dst


IB = 16
NB = CPW // IB
NPD = 10240
CPW3 = 3 * NCH // NW
RPT3 = 3 * NPD // 16


def _zero_vec16(ref):
    z = jnp.zeros((16,), jnp.float32)
    for r in range(16):
        for k in range(ref.shape[1] // 16):
            ref[r, pl.ds(k * 16, 16)] = z


@functools.partial(
    pl.kernel,
    out_type=(jax.ShapeDtypeStruct((3 * NPD,), jnp.float32),
              jax.ShapeDtypeStruct((3 * NPD,), jnp.float32)),
    mesh=_mesh,
    scratch_types=dict(
        acc=pltpu.VMEM_SHARED((3 * NPD,), jnp.float32),
        didx=pltpu.VMEM((CPW3, CH), jnp.int32),
        ones=pltpu.VMEM((CH,), jnp.float32),
        zbuf=pltpu.VMEM((((RPT3 + 15) // 16) * 16,), jnp.float32),
        sem=pltpu.SemaphoreType.DMA,
        semi=pltpu.SemaphoreType.DMA,
    ),
)
def _sc_degree(dstall, outa, outb, acc, didx, ones, zbuf, sem, semi):
    c = lax.axis_index("c")
    s = lax.axis_index("s")
    wid = s * 2 + c
    one = jnp.ones((16,), jnp.float32)
    zero = jnp.zeros((16,), jnp.float32)
    pltpu.async_copy(dstall.at[pl.ds(wid * CPW3, CPW3)], didx, semi)
    for k in range(CH // 16):
        ones[pl.ds(k * 16, 16)] = one
    for k in range((RPT3 + 15) // 16):
        zbuf[pl.ds(k * 16, 16)] = zero
    pltpu.sync_copy(zbuf.at[pl.ds(0, RPT3)], acc.at[pl.ds(s * RPT3, RPT3)])
    pltpu.make_async_copy(dstall.at[pl.ds(wid * CPW3, CPW3)], didx,
                          semi).wait()
    plsc.subcore_barrier()

    KF = 8

    def body(jj, _):
        for i in range(KF):
            pltpu.async_copy(ones, acc.at[didx.at[jj * KF + i]], sem,
                             add=True)
        for i in range(KF):
            pltpu.make_async_copy(ones, acc.at[didx.at[jj * KF + i]],
                                  sem).wait()
        return 0

    lax.fori_loop(0, CPW3 // KF, body, 0)
    plsc.subcore_barrier()

    @pl.when(c == 0)
    def _():
        pltpu.sync_copy(acc.at[pl.ds(s * RPT3, RPT3)],
                        outa.at[pl.ds(s * RPT3, RPT3)])

    @pl.when(c == 1)
    def _():
        pltpu.sync_copy(acc.at[pl.ds(s * RPT3, RPT3)],
                        outb.at[pl.ds(s * RPT3, RPT3)])


def _make_sc_agg(D):
    @functools.partial(
        pl.kernel,
        out_type=jax.ShapeDtypeStruct((2, NP, D), jnp.float32),
        mesh=_mesh,
        scratch_types=dict(
            acc=pltpu.VMEM_SHARED((NP, D), jnp.float32),
            sidx0=pltpu.VMEM((IB, CH), jnp.int32),
            sidx1=pltpu.VMEM((IB, CH), jnp.int32),
            didx0=pltpu.VMEM((IB, CH), jnp.int32),
            didx1=pltpu.VMEM((IB, CH), jnp.int32),
            rows0=pltpu.VMEM((CH, D), jnp.float32),
            rows1=pltpu.VMEM((CH, D), jnp.float32),
            semi=pltpu.SemaphoreType.DMA,
            semg0=pltpu.SemaphoreType.DMA,
            semg1=pltpu.SemaphoreType.DMA,
            sems0=pltpu.SemaphoreType.DMA,
            sems1=pltpu.SemaphoreType.DMA,
        ),
    )
    def agg(table, srcm, dstm, out, acc, sidx0, sidx1, didx0, didx1,
            rows0, rows1, semi, semg0, semg1, sems0, sems1):
        c = lax.axis_index("c")
        s = lax.axis_index("s")
        wid = s * 2 + c
        rows = (rows0, rows1)
        semg = (semg0, semg1)
        sems = (sems0, sems1)
        sidx = (sidx0, sidx1)
        didx = (didx0, didx1)
        pltpu.async_copy(srcm.at[pl.ds(wid * CPW, IB)], sidx0, semi)
        pltpu.async_copy(dstm.at[pl.ds(wid * CPW, IB)], didx0, semi)
        z = jnp.zeros((16,), jnp.float32)
        for r in range(CH):
            for k in range(D // 16):
                rows0[r, pl.ds(k * 16, 16)] = z
        for k in range(4):
            pltpu.async_copy(rows0, acc.at[pl.ds(s * RPT + k * CH, CH)],
                             sems0)
        pltpu.async_copy(rows0.at[pl.ds(0, RPT - 4 * CH)],
                         acc.at[pl.ds(s * RPT + 4 * CH, RPT - 4 * CH)],
                         sems0)
        for k in range(4):
            pltpu.make_async_copy(rows0, acc.at[pl.ds(0, CH)], sems0).wait()
        pltpu.make_async_copy(rows0.at[pl.ds(0, RPT - 4 * CH)],
                              acc.at[pl.ds(0, RPT - 4 * CH)], sems0).wait()
        pltpu.make_async_copy(srcm.at[pl.ds(wid * CPW, IB)], sidx0,
                              semi).wait()
        pltpu.make_async_copy(dstm.at[pl.ds(wid * CPW, IB)], didx0,
                              semi).wait()
        plsc.subcore_barrier()

        def drain_s(p):
            pltpu.make_async_copy(rows[p], acc.at[didx0.at[0]],
                                  sems[p]).wait()

        def wait_g(p, si, q):
            pltpu.make_async_copy(table.at[si.at[q]], rows[p],
                                  semg[p]).wait()

        pltpu.async_copy(table.at[sidx0.at[0]], rows0, semg0)

        for bb in range(NB):
            pb = bb % 2
            nb = 1 - pb
            si, di = sidx[pb], didx[pb]
            if bb + 1 < NB:
                off = wid * CPW + (bb + 1) * IB
                pltpu.async_copy(srcm.at[pl.ds(off, IB)], sidx[nb], semi)
                pltpu.async_copy(dstm.at[pl.ds(off, IB)], didx[nb], semi)

            def body(jj, _, si=si, di=di, first=(bb == 0)):
                for p in range(2):
                    q = jj * 2 + p

                    def drain(p=p):
                        drain_s(1 - p)

                    def prefetch(p=p, q=q, si=si):
                        pltpu.async_copy(table.at[si.at[q + 1]],
                                         rows[1 - p], semg[1 - p])

                    if p == 0:
                        if first:
                            pl.when(jj > 0)(drain)
                        else:
                            drain()
                        prefetch()
                    else:
                        drain()
                        prefetch()
                    wait_g(p, si, q)
                    pltpu.async_copy(rows[p], acc.at[di.at[q]], sems[p],
                                     add=True)
                return 0

            lax.fori_loop(0, IB // 2 - 1, body, 0)

            drain_s(1)
            pltpu.async_copy(table.at[si.at[IB - 1]], rows1, semg1)
            wait_g(0, si, IB - 2)
            pltpu.async_copy(rows0, acc.at[di.at[IB - 2]], sems0, add=True)

            drain_s(0)
            if bb + 1 < NB:
                off = wid * CPW + (bb + 1) * IB
                pltpu.make_async_copy(srcm.at[pl.ds(off, IB)], sidx[nb],
                                      semi).wait()
                pltpu.make_async_copy(dstm.at[pl.ds(off, IB)], didx[nb],
                                      semi).wait()
                pltpu.async_copy(table.at[sidx[nb].at[0]], rows0, semg0)
            wait_g(1, si, IB - 1)
            pltpu.async_copy(rows1, acc.at[di.at[IB - 1]], sems1, add=True)

        drain_s(1)
        plsc.subcore_barrier()
        pltpu.sync_copy(acc.at[pl.ds(s * RPT, RPT)],
                        out.at[c, pl.ds(s * RPT, RPT)])

    return agg


_sc_agg128 = _make_sc_agg(128)


_BLK = 1264
_GRID = NP // _BLK


def _dinv_body(ca_ref, cb_ref, out_ref):
    deg = ca_ref[...] + cb_ref[...] + 1.0
    out_ref[...] = lax.rsqrt(deg[:, :NP]).reshape(3, NP, 1)


def _dinv(ca, cb):
    return pl.pallas_call(
        _dinv_body,
        out_shape=jax.ShapeDtypeStruct((3, NP, 1), jnp.float32),
    )(ca.reshape(3, NPD), cb.reshape(3, NPD))


def _scale_body(dinv_ref, x_ref, out_ref):
    out_ref[...] = dinv_ref[...] * x_ref[...]


def _scale_rows(dinv, x):
    D = x.shape[1]
    return pl.pallas_call(
        _scale_body,
        grid=(_GRID,),
        in_specs=[
            pl.BlockSpec((_BLK, 1), lambda i: (i, 0)),
            pl.BlockSpec((_BLK, D), lambda i: (i, 0)),
        ],
        out_specs=pl.BlockSpec((_BLK, D), lambda i: (i, 0)),
        out_shape=jax.ShapeDtypeStruct((NP, D), jnp.float32),
    )(dinv, x)


def _layer_body(agg_ref, yp_ref, dinv_ref, wa_ref, ba_ref, wb_ref, out_ref):
    dinv = dinv_ref[...]
    t = dinv * (agg_ref[0] + agg_ref[1] + yp_ref[...])
    h = jax.nn.relu(jnp.dot(t, wa_ref[...],
                            preferred_element_type=jnp.float32) + ba_ref[...])
    out_ref[...] = dinv * jnp.dot(h, wb_ref[...],
                                  preferred_element_type=jnp.float32)


def _layer(agg, yp, dinv, Wa, ba, Wb):
    Din = yp.shape[1]
    Dh = Wa.shape[1]
    Dout = Wb.shape[1]
    return pl.pallas_call(
        _layer_body,
        grid=(_GRID,),
        in_specs=[
            pl.BlockSpec((2, _BLK, Din), lambda i: (0, i, 0)),
            pl.BlockSpec((_BLK, Din), lambda i: (i, 0)),
            pl.BlockSpec((_BLK, 1), lambda i: (i, 0)),
            pl.BlockSpec((Din, Dh), lambda i: (0, 0)),
            pl.BlockSpec((1, Dh), lambda i: (0, 0)),
            pl.BlockSpec((Dh, Dout), lambda i: (0, 0)),
        ],
        out_specs=pl.BlockSpec((_BLK, Dout), lambda i: (i, 0)),
        out_shape=jax.ShapeDtypeStruct((NP, Dout), jnp.float32),
    )(agg, yp, dinv, Wa, ba, Wb)


def _layer2_body(agg_ref, yp_ref, dinv_ref, ba_ref, wb_ref, out_ref):
    dinv = dinv_ref[...]
    h = jax.nn.relu(dinv * (agg_ref[0] + agg_ref[1] + yp_ref[...])
                    + ba_ref[...])
    out_ref[...] = dinv * jnp.dot(h, wb_ref[...],
                                  preferred_element_type=jnp.float32)


def _layer2(agg, yp, dinv, ba, Wb):
    Din = yp.shape[1]
    Dout = Wb.shape[1]
    return pl.pallas_call(
        _layer2_body,
        grid=(_GRID,),
        in_specs=[
            pl.BlockSpec((2, _BLK, Din), lambda i: (0, i, 0)),
            pl.BlockSpec((_BLK, Din), lambda i: (i, 0)),
            pl.BlockSpec((_BLK, 1), lambda i: (i, 0)),
            pl.BlockSpec((1, Din), lambda i: (0, 0)),
            pl.BlockSpec((Din, Dout), lambda i: (0, 0)),
        ],
        out_specs=pl.BlockSpec((_BLK, Dout), lambda i: (i, 0)),
        out_shape=jax.ShapeDtypeStruct((NP, Dout), jnp.float32),
    )(agg, yp, dinv, ba, Wb)


def _final_body(agg_ref, yp_ref, dinv_ref, b_ref, out_ref):
    i = pl.program_id(0)
    o = jax.nn.relu(dinv_ref[...] * (agg_ref[0] + agg_ref[1] + yp_ref[...])
                    + b_ref[...])
    row = i * _BLK + lax.broadcasted_iota(jnp.int32, (_BLK, 1), 0)
    o = jnp.where(row < N, o, 0.0)
    part = jnp.sum(o, axis=0, keepdims=True)

    @pl.when(i == 0)
    def _():
        out_ref[...] = jnp.zeros_like(out_ref)

    out_ref[...] += part


def _final(agg, yp, dinv, b3):
    Dout = yp.shape[1]
    return pl.pallas_call(
        _final_body,
        grid=(_GRID,),
        in_specs=[
            pl.BlockSpec((2, _BLK, Dout), lambda i: (0, i, 0)),
            pl.BlockSpec((_BLK, Dout), lambda i: (i, 0)),
            pl.BlockSpec((_BLK, 1), lambda i: (i, 0)),
            pl.BlockSpec((1, Dout), lambda i: (0, 0)),
        ],
        out_specs=pl.BlockSpec((1, Dout), lambda i: (0, 0)),
        out_shape=jax.ShapeDtypeStruct((1, Dout), jnp.float32),
    )(agg, yp, dinv, b3)


def kernel(x1, edge_index1, x2, edge_index2, x3, edge_index3,
           W1, b1, W2, b2, W3, b3):
    b1r = b1.reshape(1, -1)
    b2r = b2.reshape(1, -1)
    W3p = jnp.pad(W3, ((0, 0), (0, 64)))
    b3r = jnp.pad(b3, (0, 64)).reshape(1, -1)
    pads = [_pad_edges(ei) for ei in (edge_index1, edge_index2, edge_index3)]
    dstall = jnp.concatenate(
        [pads[b][1].reshape(-1) + b * NPD for b in range(3)]
    ).reshape(3 * NCH, CH)
    ca, cb = _sc_degree(dstall)
    dinv = _dinv(ca, cb)

    total = jnp.zeros((1, 128), jnp.float32)
    for bidx, x in enumerate((x1, x2, x3)):
        src, dst = pads[bidx]
        dv = lax.slice(dinv, (bidx, 0, 0), (bidx + 1, NP, 1)).reshape(NP, 1)
        xp = jnp.pad(x, ((0, NP - N), (0, 0)))
        xs = _scale_rows(dv, xp)
        agg1 = _sc_agg128(xs, src, dst)
        yp2 = _layer(agg1, xs, dv, W1, b1r, W2)
        agg2 = _sc_agg128(yp2, src, dst)
        yp3 = _layer2(agg2, yp2, dv, b2r, W3p)
        agg3 = _sc_agg128(yp3, src, dst)
        total = total + _final(agg3, yp3, dv, b3r)
    return jnp.sum(total) / 64.0
